# baseline (speedup 1.0000x reference)
import functools

import jax
import jax.numpy as jnp
from jax import lax
from jax.experimental import pallas as pl
from jax.experimental.pallas import tpu as pltpu
from jax.experimental.pallas import tpu_sc as plsc

NUM_EXPERTS = 64
TOP_K = 8
LANES = 16
NUM_CORES = 2
NUM_SUBCORES = 16
NUM_WORKERS = NUM_CORES * NUM_SUBCORES
NUM_CHUNKS = 4
UNROLL = 4
RPAD = 129


def _tec_kernel(T, ST, conf_hbm, wealth_hbm, eidx_hbm, wgt_hbm, pay_hbm,
                conf_v0, conf_v1, w_v, eidx_v, wgt_v, pay_v, sem0, sem1):
  wid = lax.axis_index("s") * NUM_CORES + lax.axis_index("c")
  S = ST * 128
  wpb = S // T
  b = wid // wpb
  s0 = (wid % wpb) * T
  tc0 = s0 // 128

  pltpu.sync_copy(wealth_hbm, w_v)

  w0 = w_v[pl.ds(0, LANES)]
  w1 = w_v[pl.ds(16, LANES)]
  w2 = w_v[pl.ds(32, LANES)]
  w3 = w_v[pl.ds(48, LANES)]

  iota = lax.iota(jnp.int32, LANES)
  idx0 = iota
  idx1 = iota + 16
  idx2 = iota + 32
  idx3 = iota + 48
  lane_lt8 = iota < TOP_K
  k_vec = iota & 7

  C = T // NUM_CHUNKS
  CT = C // 128
  RPC = 8 * CT * 8
  r_base = (iota >> 3) * (CT * 8) + (iota & 7)
  r0 = r_base
  r1 = r_base + 2 * (CT * 8)
  r2 = r_base + 4 * (CT * 8)
  r3 = r_base + 6 * (CT * 8)

  def merge_top16(ak, av, bk, bv):
    rbk = lax.rev(bk, (0,))
    rbv = lax.rev(bv, (0,))
    c = ak >= rbk
    return jnp.where(c, ak, rbk), jnp.where(c, av, rbv)

  def body(t, coff, conf_v):
    tr = jnp.full_like(iota, (t >> 7) * 8)
    tl_vec = jnp.full_like(iota, t & 127)
    b0 = plsc.load_gather(conf_v, [r0 + tr, tl_vec]) * w0
    b1 = plsc.load_gather(conf_v, [r1 + tr, tl_vec]) * w1
    b2 = plsc.load_gather(conf_v, [r2 + tr, tl_vec]) * w2
    b3 = plsc.load_gather(conf_v, [r3 + tr, tl_vec]) * w3

    s0k, s0v = plsc.sort_key_val(b0, idx0, descending=True)
    s1k, s1v = plsc.sort_key_val(b1, idx1, descending=True)
    s2k, s2v = plsc.sort_key_val(b2, idx2, descending=True)
    s3k, s3v = plsc.sort_key_val(b3, idx3, descending=True)

    h01k, h01v = merge_top16(s0k, s0v, s1k, s1v)
    h23k, h23v = merge_top16(s2k, s2v, s3k, s3v)
    m01k, m01v = plsc.sort_key_val(h01k, h01v, descending=True)
    m23k, m23v = plsc.sort_key_val(h23k, h23v, descending=True)
    hk, hv = merge_top16(m01k, m01v, m23k, m23v)
    fk, fv = plsc.sort_key_val(hk, hv, descending=True)

    pay = jnp.max(jnp.where(lane_lt8, -1.0, fk))
    e = jnp.where(lane_lt8, jnp.exp(fk), 0.0)
    wgt = e / jnp.sum(e)

    tl = coff + t
    oidx = [jnp.full_like(iota, tl >> 7), k_vec,
            jnp.full_like(iota, tl & 127)]
    plsc.store_scatter(eidx_v, oidx, fv, mask=lane_lt8)
    plsc.store_scatter(wgt_v, oidx, wgt, mask=lane_lt8)
    plsc.store_scatter(pay_v, oidx, jnp.where(lane_lt8, pay, 0.0),
                       mask=lane_lt8)

  bufs = (conf_v0, conf_v1)
  sems = (sem0, sem1)

  def copies(ch):
    out = []
    for eg in range(8):
      row0 = ((b * 8 + eg) * ST + tc0 + ch * CT) * 8
      out.append(pltpu.make_async_copy(
          conf_hbm.at[pl.ds(row0, CT * 8)],
          bufs[ch % 2].at[pl.ds(eg * CT * 8, CT * 8), pl.ds(0, 128)],
          sems[ch % 2]))
    return out

  for cp in copies(0):
    cp.start()
  for ch in range(NUM_CHUNKS):
    if ch + 1 < NUM_CHUNKS:
      for cp in copies(ch + 1):
        cp.start()
    for cp in copies(ch):
      cp.wait()
    plsc.parallel_loop(0, C, unroll=UNROLL)(
        lambda t, coff=ch * C, cv=bufs[ch % 2]: body(t, coff, cv))

  nt = T // 128
  pltpu.sync_copy(eidx_v, eidx_hbm.at[b, pl.ds(tc0, nt)])
  pltpu.sync_copy(wgt_v, wgt_hbm.at[b, pl.ds(tc0, nt)])
  pltpu.sync_copy(pay_v, pay_hbm.at[b, pl.ds(tc0, nt)])


@jax.jit
def kernel(confidences, wealth):
  B, S, E = confidences.shape
  N = B * S
  T = N // NUM_WORKERS
  NT = T // 128
  ST = S // 128
  conf_t = confidences.reshape(B, ST, 128, E // 8, 8)
  conf_t = conf_t.transpose(0, 3, 1, 4, 2).reshape(-1, 128)

  mesh = plsc.VectorSubcoreMesh(
      core_axis_name="c", subcore_axis_name="s",
      num_cores=NUM_CORES, num_subcores=NUM_SUBCORES)

  RPC = E * (T // NUM_CHUNKS) // 128

  eidx, wgt, pay = pl.kernel(
      functools.partial(_tec_kernel, T, ST),
      out_type=(
          jax.ShapeDtypeStruct((B, ST, TOP_K, 128), jnp.int32),
          jax.ShapeDtypeStruct((B, ST, TOP_K, 128), jnp.float32),
          jax.ShapeDtypeStruct((B, ST, TOP_K, 128), jnp.float32),
      ),
      mesh=mesh,
      compiler_params=pltpu.CompilerParams(needs_layout_passes=False),
      scratch_types=[
          pltpu.VMEM((RPC, RPAD), jnp.float32),
          pltpu.VMEM((RPC, RPAD), jnp.float32),
          pltpu.VMEM((E,), jnp.float32),
          pltpu.VMEM((NT, TOP_K, 128), jnp.int32),
          pltpu.VMEM((NT, TOP_K, 128), jnp.float32),
          pltpu.VMEM((NT, TOP_K, 128), jnp.float32),
          pltpu.SemaphoreType.DMA,
          pltpu.SemaphoreType.DMA,
      ],
  )(conf_t, wealth)

  def detile(x):
    return x.transpose(0, 1, 3, 2).reshape(B, S, TOP_K)

  return (detile(eidx), detile(wgt), detile(pay))


# speedup vs baseline: 1.3161x; 1.3161x over previous
import functools

import jax
import jax.numpy as jnp
from jax import lax
from jax.experimental import pallas as pl
from jax.experimental.pallas import tpu as pltpu
from jax.experimental.pallas import tpu_sc as plsc

NUM_EXPERTS = 64
TOP_K = 8
LANES = 16
NUM_CORES = 2
NUM_SUBCORES = 16
NUM_WORKERS = NUM_CORES * NUM_SUBCORES
NUM_CHUNKS = 4
UNROLL = 4


def _tec_kernel(T, conf_hbm, wealth_hbm, eidx_hbm, wgt_hbm, pay_hbm,
                conf_v0, conf_v1, w_v, eidx_v, wgt_v, pay_v, sem0, sem1):
  wid = lax.axis_index("s") * NUM_CORES + lax.axis_index("c")
  S = conf_hbm.shape[1]
  wpb = S // T
  b = wid // wpb
  s0 = (wid % wpb) * T
  tc0 = s0 // 128

  pltpu.sync_copy(wealth_hbm, w_v)

  w0 = w_v[pl.ds(0, LANES)]
  w1 = w_v[pl.ds(16, LANES)]
  w2 = w_v[pl.ds(32, LANES)]
  w3 = w_v[pl.ds(48, LANES)]

  iota = lax.iota(jnp.int32, LANES)
  idx0 = iota
  idx1 = iota + 16
  idx2 = iota + 32
  idx3 = iota + 48
  lane_lt8 = iota < TOP_K
  k128 = (iota & 7) * 128

  def merge_top16(ak, av, bk, bv):
    rbk = lax.rev(bk, (0,))
    rbv = lax.rev(bv, (0,))
    c = ak >= rbk
    return jnp.where(c, ak, rbk), jnp.where(c, av, rbv)

  def body(t, coff, conf_v):
    b0 = conf_v[t, pl.ds(0, LANES)] * w0
    b1 = conf_v[t, pl.ds(16, LANES)] * w1
    b2 = conf_v[t, pl.ds(32, LANES)] * w2
    b3 = conf_v[t, pl.ds(48, LANES)] * w3

    s0k, s0v = plsc.sort_key_val(b0, idx0, descending=True)
    s1k, s1v = plsc.sort_key_val(b1, idx1, descending=True)
    s2k, s2v = plsc.sort_key_val(b2, idx2, descending=True)
    s3k, s3v = plsc.sort_key_val(b3, idx3, descending=True)

    h01k, h01v = merge_top16(s0k, s0v, s1k, s1v)
    h23k, h23v = merge_top16(s2k, s2v, s3k, s3v)
    m01k, m01v = plsc.sort_key_val(h01k, h01v, descending=True)
    m23k, m23v = plsc.sort_key_val(h23k, h23v, descending=True)
    hk, hv = merge_top16(m01k, m01v, m23k, m23v)
    fk, fv = plsc.sort_key_val(hk, hv, descending=True)

    pay = jnp.max(jnp.where(lane_lt8, -1.0, fk))
    e = jnp.where(lane_lt8, jnp.exp(fk), 0.0)
    wgt = e / jnp.sum(e)

    tl = coff + t
    oidx = [k128 + ((tl >> 7) * (TOP_K * 128) + (tl & 127))]
    plsc.store_scatter(eidx_v, oidx, fv, mask=lane_lt8)
    plsc.store_scatter(wgt_v, oidx, wgt, mask=lane_lt8)
    plsc.store_scatter(pay_v, oidx, jnp.full_like(fk, pay), mask=lane_lt8)

  C = T // NUM_CHUNKS
  bufs = (conf_v0, conf_v1)
  sems = (sem0, sem1)

  def copy(ch):
    return pltpu.make_async_copy(
        conf_hbm.at[b, pl.ds(s0 + ch * C, C)], bufs[ch % 2], sems[ch % 2])

  copy(0).start()
  for ch in range(NUM_CHUNKS):
    if ch + 1 < NUM_CHUNKS:
      copy(ch + 1).start()
    copy(ch).wait()
    plsc.parallel_loop(0, C, unroll=UNROLL)(
        lambda t, coff=ch * C, cv=bufs[ch % 2]: body(t, coff, cv))

  ST = S // 128
  off = (b * ST + tc0) * (TOP_K * 128)
  sz = T * TOP_K
  pltpu.sync_copy(eidx_v, eidx_hbm.at[pl.ds(off, sz)])
  pltpu.sync_copy(wgt_v, wgt_hbm.at[pl.ds(off, sz)])
  pltpu.sync_copy(pay_v, pay_hbm.at[pl.ds(off, sz)])


@jax.jit
def kernel(confidences, wealth):
  B, S, E = confidences.shape
  N = B * S
  T = N // NUM_WORKERS
  NT = T // 128

  mesh = plsc.VectorSubcoreMesh(
      core_axis_name="c", subcore_axis_name="s",
      num_cores=NUM_CORES, num_subcores=NUM_SUBCORES)

  eidx, wgt, pay = pl.kernel(
      functools.partial(_tec_kernel, T),
      out_type=(
          jax.ShapeDtypeStruct((N * TOP_K,), jnp.int32),
          jax.ShapeDtypeStruct((N * TOP_K,), jnp.float32),
          jax.ShapeDtypeStruct((N * TOP_K,), jnp.float32),
      ),
      mesh=mesh,
      compiler_params=pltpu.CompilerParams(needs_layout_passes=False),
      scratch_types=[
          pltpu.VMEM((T // NUM_CHUNKS, E), jnp.float32),
          pltpu.VMEM((T // NUM_CHUNKS, E), jnp.float32),
          pltpu.VMEM((E,), jnp.float32),
          pltpu.VMEM((T * TOP_K,), jnp.int32),
          pltpu.VMEM((T * TOP_K,), jnp.float32),
          pltpu.VMEM((T * TOP_K,), jnp.float32),
          pltpu.SemaphoreType.DMA,
          pltpu.SemaphoreType.DMA,
      ],
  )(confidences, wealth)

  def detile(x):
    x = x.reshape(B, S // 128, TOP_K, 128)
    return x.transpose(0, 1, 3, 2).reshape(B, S, TOP_K)

  return (detile(eidx), detile(wgt), detile(pay))
